# baseline (speedup 1.0000x reference)
import functools

import jax
import jax.numpy as jnp
from jax.experimental import pallas as pl
from jax.experimental.pallas import tpu as pltpu


def _main_kernel(ct_ref, cs_ref, fta_ref, ftb_ref, fsa_ref, fsb_ref,
                 pool_ref, out_ref, num_acc, den_acc,
                 *, nj, tb, c, g, nstream, inv_d, loss_weight):
    j = pl.program_id(0)

    @pl.when(j == 0)
    def _init():
        num_acc[...] = jnp.zeros_like(num_acc)
        den_acc[...] = jnp.zeros_like(den_acc)

    d = jax.nn.sigmoid(ct_ref[...]) - jax.nn.sigmoid(cs_ref[...])
    d2 = d * d
    s = jnp.sum(d2.reshape(tb, c, d2.shape[-1]), axis=1)
    w = jnp.dot(s, pool_ref[...],
                preferred_element_type=jnp.float32)

    halves = ((fta_ref, fsa_ref), (ftb_ref, fsb_ref))[:nstream]
    bh = tb // nstream
    for h, (ft_ref, fs_ref) in enumerate(halves):
        e = fs_ref[...] - ft_ref[...]
        e2 = e * e
        for b in range(bh):
            wb = w[h * bh + b:h * bh + b + 1, :]
            num_acc[...] += jnp.dot(wb, e2[b * g:(b + 1) * g, :],
                                    preferred_element_type=jnp.float32)
            den_acc[...] += wb

    @pl.when(j == nj - 1)
    def _finalize():
        num = jnp.sum(num_acc[...])
        den = jnp.sum(den_acc[...])
        out_ref[0, 0] = num * inv_d / den * loss_weight


def kernel(conf_t, feature_t, conf_s, feature_s):
    loss_weight = 5.0
    B, A, C = conf_t.shape
    G = A // 6
    D = feature_t.shape[-1]

    ct = jnp.swapaxes(conf_t, 1, 2).reshape(B * C, A)
    cs = jnp.swapaxes(conf_s, 1, 2).reshape(B * C, A)
    ft = feature_t.reshape(B * G, D)
    fs = feature_s.reshape(B * G, D)

    nj = next(n for n in (2, 1) if B % n == 0)
    tb = B // nj
    nstream = 2 if tb % 2 == 0 else 1
    bh = tb // nstream

    pool = (jnp.arange(A, dtype=jnp.int32)[:, None] // 6 ==
            jnp.arange(G, dtype=jnp.int32)[None, :]).astype(jnp.float32)

    def fea_idx(k):
        return lambda j, k=k: (nstream * j + k, 0)

    out = pl.pallas_call(
        functools.partial(_main_kernel, nj=nj, tb=tb, c=C, g=G,
                          nstream=nstream, inv_d=1.0 / float(D),
                          loss_weight=float(loss_weight)),
        out_shape=jax.ShapeDtypeStruct((1, 1), jnp.float32),
        grid=(nj,),
        in_specs=[
            pl.BlockSpec((tb * C, A), lambda j: (j, 0)),
            pl.BlockSpec((tb * C, A), lambda j: (j, 0)),
            pl.BlockSpec((bh * G, D), fea_idx(0)),
            pl.BlockSpec((bh * G, D), fea_idx(nstream - 1)),
            pl.BlockSpec((bh * G, D), fea_idx(0)),
            pl.BlockSpec((bh * G, D), fea_idx(nstream - 1)),
            pl.BlockSpec((A, G), lambda j: (0, 0)),
        ],
        out_specs=pl.BlockSpec((1, 1), lambda j: (0, 0),
                               memory_space=pltpu.SMEM),
        scratch_shapes=[pltpu.VMEM((1, D), jnp.float32),
                        pltpu.VMEM((1, G), jnp.float32)],
        compiler_params=pltpu.CompilerParams(
            dimension_semantics=("arbitrary",),
            vmem_limit_bytes=100 * 1024 * 1024),
    )(ct, cs, ft, ft, fs, fs, pool)
    return out[0, 0]


# speedup vs baseline: 1.0431x; 1.0431x over previous
import functools

import jax
import jax.numpy as jnp
from jax.experimental import pallas as pl
from jax.experimental.pallas import tpu as pltpu

_NJ = 4
_FSPLIT = 4
_CSPLIT = 2


def _main_kernel(*refs, nj, tb, c, g, fsplit, csplit, inv_d, loss_weight):
    ct_refs = refs[:csplit]
    cs_refs = refs[csplit:2 * csplit]
    ft_refs = refs[2 * csplit:2 * csplit + fsplit]
    fs_refs = refs[2 * csplit + fsplit:2 * csplit + 2 * fsplit]
    pool_ref, out_ref, num_acc, den_acc = refs[2 * csplit + 2 * fsplit:]
    j = pl.program_id(0)

    @pl.when(j == 0)
    def _init():
        num_acc[...] = jnp.zeros_like(num_acc)
        den_acc[...] = jnp.zeros_like(den_acc)

    tbc = tb // csplit
    ws = []
    for ct_ref, cs_ref in zip(ct_refs, cs_refs):
        d = jax.nn.sigmoid(ct_ref[...]) - jax.nn.sigmoid(cs_ref[...])
        d2 = d * d
        s = jnp.sum(d2.reshape(tbc, c, d2.shape[-1]), axis=1)
        ws.append(jnp.dot(s, pool_ref[...],
                          preferred_element_type=jnp.float32))

    bh = tb // fsplit
    for h, (ft_ref, fs_ref) in enumerate(zip(ft_refs, fs_refs)):
        e = fs_ref[...] - ft_ref[...]
        e2 = e * e
        for b in range(bh):
            babs = h * bh + b
            wb = ws[babs // tbc][babs % tbc:babs % tbc + 1, :]
            num_acc[...] += jnp.dot(wb, e2[b * g:(b + 1) * g, :],
                                    preferred_element_type=jnp.float32)
            den_acc[...] += wb

    @pl.when(j == nj - 1)
    def _finalize():
        num = jnp.sum(num_acc[...])
        den = jnp.sum(den_acc[...])
        out_ref[0, 0] = num * inv_d / den * loss_weight


def kernel(conf_t, feature_t, conf_s, feature_s):
    loss_weight = 5.0
    B, A, C = conf_t.shape
    G = A // 6
    D = feature_t.shape[-1]

    ct = jnp.swapaxes(conf_t, 1, 2).reshape(B * C, A)
    cs = jnp.swapaxes(conf_s, 1, 2).reshape(B * C, A)
    ft = feature_t.reshape(B * G, D)
    fs = feature_s.reshape(B * G, D)

    nj = next(n for n in (_NJ, 2, 1) if B % n == 0)
    tb = B // nj
    fsplit = next(f for f in (_FSPLIT, 2, 1) if tb % f == 0)
    csplit = next(f for f in (_CSPLIT, 1) if tb % f == 0)
    bh = tb // fsplit
    tbc = tb // csplit

    pool = (jnp.arange(A, dtype=jnp.int32)[:, None] // 6 ==
            jnp.arange(G, dtype=jnp.int32)[None, :]).astype(jnp.float32)

    def idx2(split, k):
        return lambda j, k=k, s=split: (s * j + k, 0)

    in_specs = (
        [pl.BlockSpec((tbc * C, A), idx2(csplit, k)) for k in range(csplit)] +
        [pl.BlockSpec((tbc * C, A), idx2(csplit, k)) for k in range(csplit)] +
        [pl.BlockSpec((bh * G, D), idx2(fsplit, k)) for k in range(fsplit)] +
        [pl.BlockSpec((bh * G, D), idx2(fsplit, k)) for k in range(fsplit)] +
        [pl.BlockSpec((A, G), lambda j: (0, 0))]
    )
    operands = ([ct] * csplit + [cs] * csplit +
                [ft] * fsplit + [fs] * fsplit + [pool])

    out = pl.pallas_call(
        functools.partial(_main_kernel, nj=nj, tb=tb, c=C, g=G,
                          fsplit=fsplit, csplit=csplit, inv_d=1.0 / float(D),
                          loss_weight=float(loss_weight)),
        out_shape=jax.ShapeDtypeStruct((1, 1), jnp.float32),
        grid=(nj,),
        in_specs=in_specs,
        out_specs=pl.BlockSpec((1, 1), lambda j: (0, 0),
                               memory_space=pltpu.SMEM),
        scratch_shapes=[pltpu.VMEM((1, D), jnp.float32),
                        pltpu.VMEM((1, G), jnp.float32)],
        compiler_params=pltpu.CompilerParams(
            dimension_semantics=("arbitrary",),
            vmem_limit_bytes=100 * 1024 * 1024),
    )(*operands)
    return out[0, 0]


# speedup vs baseline: 1.0541x; 1.0105x over previous
import functools

import jax
import jax.numpy as jnp
from jax.experimental import pallas as pl
from jax.experimental.pallas import tpu as pltpu

_NJ = 4
_FSPLIT = 2
_CSPLIT = 1


def _main_kernel(*refs, nj, tb, c, g, fsplit, csplit, inv_d, loss_weight):
    ct_refs = refs[:csplit]
    cs_refs = refs[csplit:2 * csplit]
    ft_refs = refs[2 * csplit:2 * csplit + fsplit]
    fs_refs = refs[2 * csplit + fsplit:2 * csplit + 2 * fsplit]
    pool_ref, out_ref, num_acc, den_acc = refs[2 * csplit + 2 * fsplit:]
    j = pl.program_id(0)

    @pl.when(j == 0)
    def _init():
        num_acc[...] = jnp.zeros_like(num_acc)
        den_acc[...] = jnp.zeros_like(den_acc)

    tbc = tb // csplit
    ws = []
    for ct_ref, cs_ref in zip(ct_refs, cs_refs):
        d = jax.nn.sigmoid(ct_ref[...]) - jax.nn.sigmoid(cs_ref[...])
        d2 = d * d
        s = jnp.sum(d2.reshape(tbc, c, d2.shape[-1]), axis=1)
        ws.append(jnp.dot(s, pool_ref[...],
                          preferred_element_type=jnp.float32))

    bh = tb // fsplit
    for h, (ft_ref, fs_ref) in enumerate(zip(ft_refs, fs_refs)):
        e = fs_ref[...] - ft_ref[...]
        e2 = e * e
        for b in range(bh):
            babs = h * bh + b
            wb = ws[babs // tbc][babs % tbc:babs % tbc + 1, :]
            num_acc[...] += jnp.dot(wb, e2[b * g:(b + 1) * g, :],
                                    preferred_element_type=jnp.float32)
            den_acc[...] += wb

    @pl.when(j == nj - 1)
    def _finalize():
        num = jnp.sum(num_acc[...])
        den = jnp.sum(den_acc[...])
        out_ref[0, 0] = num * inv_d / den * loss_weight


def kernel(conf_t, feature_t, conf_s, feature_s):
    loss_weight = 5.0
    B, A, C = conf_t.shape
    G = A // 6
    D = feature_t.shape[-1]

    ct = jnp.swapaxes(conf_t, 1, 2).reshape(B * C, A)
    cs = jnp.swapaxes(conf_s, 1, 2).reshape(B * C, A)
    ft = feature_t.reshape(B * G, D)
    fs = feature_s.reshape(B * G, D)

    nj = next(n for n in (_NJ, 2, 1) if B % n == 0)
    tb = B // nj
    fsplit = next(f for f in (_FSPLIT, 2, 1) if tb % f == 0)
    csplit = next(f for f in (_CSPLIT, 1) if tb % f == 0)
    bh = tb // fsplit
    tbc = tb // csplit

    pool = (jnp.arange(A, dtype=jnp.int32)[:, None] // 6 ==
            jnp.arange(G, dtype=jnp.int32)[None, :]).astype(jnp.float32)

    def idx2(split, k):
        return lambda j, k=k, s=split: (s * j + k, 0)

    in_specs = (
        [pl.BlockSpec((tbc * C, A), idx2(csplit, k)) for k in range(csplit)] +
        [pl.BlockSpec((tbc * C, A), idx2(csplit, k)) for k in range(csplit)] +
        [pl.BlockSpec((bh * G, D), idx2(fsplit, k)) for k in range(fsplit)] +
        [pl.BlockSpec((bh * G, D), idx2(fsplit, k)) for k in range(fsplit)] +
        [pl.BlockSpec((A, G), lambda j: (0, 0))]
    )
    operands = ([ct] * csplit + [cs] * csplit +
                [ft] * fsplit + [fs] * fsplit + [pool])

    out = pl.pallas_call(
        functools.partial(_main_kernel, nj=nj, tb=tb, c=C, g=G,
                          fsplit=fsplit, csplit=csplit, inv_d=1.0 / float(D),
                          loss_weight=float(loss_weight)),
        out_shape=jax.ShapeDtypeStruct((1, 1), jnp.float32),
        grid=(nj,),
        in_specs=in_specs,
        out_specs=pl.BlockSpec((1, 1), lambda j: (0, 0),
                               memory_space=pltpu.SMEM),
        scratch_shapes=[pltpu.VMEM((1, D), jnp.float32),
                        pltpu.VMEM((1, G), jnp.float32)],
        compiler_params=pltpu.CompilerParams(
            dimension_semantics=("arbitrary",),
            vmem_limit_bytes=100 * 1024 * 1024),
    )(*operands)
    return out[0, 0]
